# Initial kernel scaffold; baseline (speedup 1.0000x reference)
import jax
import jax.numpy as jnp
from jax.experimental import pallas as pl


def kernel(x, edge_index, W1a, b1a, g1a, be1a, W2a, b2a, W1b, b1b, g1b, be1b, W2b, b2b):
    raise NotImplementedError("write your pallas kernel here")


# speedup vs baseline: 8.3734x; 8.3734x over previous
import functools

import jax
import jax.numpy as jnp
from jax import lax
from jax.experimental import pallas as pl
from jax.experimental.pallas import tpu as pltpu
from jax.experimental.pallas import tpu_sc as plsc

N = 10000
E = 320000
NFEAT = 128
EPS = 1e-7

CH = 128
TILES = 16
CPT = -(-E // (CH * TILES))
CHUNKS = CPT * TILES
E_PAD = CHUNKS * CH
NACC = ((N // TILES) + (0 if N % TILES == 0 else 1)) * TILES
NACC = 10240
ZROWS = NACC // TILES
OROWS = N // TILES
DUMMY = N


def _edge_accumulate(p_tbl, q_tbl, src2d, dst2d):
    mesh = plsc.VectorSubcoreMesh(core_axis_name="c", subcore_axis_name="s")

    @functools.partial(
        pl.kernel,
        out_type=(
            jax.ShapeDtypeStruct((NACC, NFEAT), jnp.float32),
            jax.ShapeDtypeStruct((NACC, NFEAT), jnp.float32),
        ),
        mesh=mesh,
        scratch_types=[
            pltpu.VMEM((CH,), jnp.int32),
            pltpu.VMEM((CH,), jnp.int32),
            pltpu.VMEM((CH, NFEAT), jnp.float32),
            pltpu.VMEM_SHARED((NACC, NFEAT), jnp.float32),
            pltpu.SemaphoreType.DMA,
        ],
    )
    def k(p_ref, q_ref, src_ref, dst_ref, op_ref, oq_ref,
          src_v, dst_v, rows_v, acc, sem):
        cid = lax.axis_index("c")
        sid = lax.axis_index("s")

        zero = jnp.zeros((16,), jnp.float32)

        def zero_row(r, carry):
            for kk in range(NFEAT // 16):
                rows_v[r, pl.ds(kk * 16, 16)] = zero
            return carry

        lax.fori_loop(0, CH, zero_row, 0)
        for b in range(ZROWS // CH):
            pltpu.sync_copy(rows_v, acc.at[pl.ds(sid * ZROWS + b * CH, CH)])
        plsc.subcore_barrier()

        def run(tbl, out):
            base = sid * CPT

            def body(j, carry):
                pltpu.sync_copy(src_ref.at[base + j], src_v)
                pltpu.sync_copy(dst_ref.at[base + j], dst_v)
                pltpu.async_copy(tbl.at[src_v], rows_v, sem).wait()
                pltpu.sync_copy(rows_v, acc.at[dst_v], add=True)
                return carry

            lax.fori_loop(0, CPT, body, 0)
            plsc.subcore_barrier()
            pltpu.sync_copy(acc.at[pl.ds(sid * ZROWS, ZROWS)],
                            out.at[pl.ds(sid * ZROWS, ZROWS)])

        @pl.when(cid == 0)
        def _():
            run(p_ref, op_ref)

        @pl.when(cid == 1)
        def _():
            run(q_ref, oq_ref)

    accp, accq = k(p_tbl, q_tbl, src2d, dst2d)
    return accp[:N], accq[:N]


def _prep(x):
    def body(x_ref, p_ref, q_ref):
        m = jnp.maximum(x_ref[:], 0.0) + EPS
        q = jnp.exp(m)
        p_ref[:] = m * q
        q_ref[:] = q

    return pl.pallas_call(
        body,
        out_shape=(
            jax.ShapeDtypeStruct((N, NFEAT), jnp.float32),
            jax.ShapeDtypeStruct((N, NFEAT), jnp.float32),
        ),
    )(x)


def _mlp(accp, accq, xin, W1, b1, g1, be1, W2, b2, *, fuse_next):
    hid2 = W1.shape[1]

    def body(ap_ref, aq_ref, x_ref, w1_ref, b1_ref, g1_ref, be1_ref,
             w2_ref, b2_ref, *outs):
        agg = ap_ref[:] / jnp.maximum(aq_ref[:], 1e-16)
        out = agg + x_ref[:]
        h = jnp.dot(out, w1_ref[:], preferred_element_type=jnp.float32) + b1_ref[:]
        mu = jnp.mean(h, axis=0, keepdims=True)
        var = jnp.mean((h - mu) ** 2, axis=0, keepdims=True)
        hn = (h - mu) * (g1_ref[:] * lax.rsqrt(var + 1e-5)) + be1_ref[:]
        hr = jnp.maximum(hn, 0.0)
        y = jnp.dot(hr, w2_ref[:], preferred_element_type=jnp.float32) + b2_ref[:]
        if fuse_next:
            x2_ref, p_ref, q_ref = outs
            x2_ref[:] = jnp.where(y >= 0, y, 0.01 * y)
            m = jnp.maximum(y, 0.0) + EPS
            q = jnp.exp(m)
            p_ref[:] = m * q
            q_ref[:] = q
        else:
            outs[0][:] = y

    nout = 3 if fuse_next else 1
    return pl.pallas_call(
        body,
        out_shape=tuple(
            jax.ShapeDtypeStruct((N, NFEAT), jnp.float32) for _ in range(nout)
        ),
    )(accp, accq, xin,
      W1, b1.reshape(1, hid2), g1.reshape(1, hid2), be1.reshape(1, hid2),
      W2, b2.reshape(1, NFEAT))


def kernel(x, edge_index, W1a, b1a, g1a, be1a, W2a, b2a,
           W1b, b1b, g1b, be1b, W2b, b2b):
    src = edge_index[0]
    dst = edge_index[1]
    pad = E_PAD - E
    src2d = jnp.concatenate(
        [src, jnp.zeros((pad,), jnp.int32)]).reshape(CHUNKS, CH)
    dst2d = jnp.concatenate(
        [dst, jnp.full((pad,), DUMMY, jnp.int32)]).reshape(CHUNKS, CH)

    p1, q1 = _prep(x)
    ap1, aq1 = _edge_accumulate(p1, q1, src2d, dst2d)
    x2, p2, q2 = _mlp(ap1, aq1, x, W1a, b1a, g1a, be1a, W2a, b2a,
                      fuse_next=True)
    ap2, aq2 = _edge_accumulate(p2, q2, src2d, dst2d)
    (y,) = _mlp(ap2, aq2, x2, W1b, b1b, g1b, be1b, W2b, b2b,
                fuse_next=False)
    return y
